# Initial kernel scaffold; baseline (speedup 1.0000x reference)
import functools

import jax
import jax.numpy as jnp
from jax import lax
from jax.experimental import pallas as pl
from jax.experimental.pallas import tpu as pltpu
from jax.experimental.pallas import tpu_sc as plsc

C = 96
H = 512
W = 512
FH = 128
FW = 128
L = FH * FW
NW = 32
PR_PER_W = FH // NW
ROWS_PER_W = 4 * PR_PER_W
LW = PR_PER_W * FW


def _body(x_hbm, s_hbm, out_hbm, sbuf, xbuf, obuf):
    cid = lax.axis_index("c")
    sid = lax.axis_index("s")
    wid = sid * 2 + cid
    base_l = wid * LW
    pltpu.sync_copy(s_hbm.at[pl.ds(base_l, LW)], sbuf)
    lane4 = lax.iota(jnp.int32, 16) * 4

    def chan(c, carry):
        pltpu.sync_copy(
            x_hbm.at[pl.ds(c * H + wid * ROWS_PER_W, ROWS_PER_W), :], xbuf
        )
        for i in range(LW // 16):
            s = sbuf[pl.ds(i * 16, 16)]
            row = (s >> 2) + ((i * 16) // FW) * 4
            col = (s & 3) + lane4 + ((i * 16) % FW) * 4
            obuf[pl.ds(i * 16, 16)] = plsc.load_gather(xbuf, [row, col])
        pltpu.sync_copy(obuf, out_hbm.at[pl.ds(c * L + base_l, LW)])
        return carry

    lax.fori_loop(0, C, chan, 0)


@jax.jit
def _run(xr, sidx):
    mesh = plsc.VectorSubcoreMesh(core_axis_name="c", subcore_axis_name="s")
    kfn = pl.kernel(
        _body,
        out_type=jax.ShapeDtypeStruct((C * L,), jnp.float32),
        mesh=mesh,
        scratch_types=[
            pltpu.VMEM((LW,), jnp.int32),
            pltpu.VMEM((ROWS_PER_W, W), jnp.float32),
            pltpu.VMEM((LW,), jnp.float32),
        ],
    )
    return kfn(xr, sidx)


def kernel(x):
    b, c, h, w = x.shape
    sidx = jax.random.randint(
        jax.random.key(42), (b, 1, 1, L), 0, 16
    ).reshape(L).astype(jnp.int32)
    xr = x.reshape(C * H, W)
    out = _run(xr, sidx)
    return out.reshape(1, C, L)


# speedup vs baseline: 1.5719x; 1.5719x over previous
import functools

import jax
import jax.numpy as jnp
from jax import lax
from jax.experimental import pallas as pl
from jax.experimental.pallas import tpu as pltpu
from jax.experimental.pallas import tpu_sc as plsc

C = 96
H = 512
W = 512
FH = 128
FW = 128
L = FH * FW
NW = 32
PR_PER_W = FH // NW
ROWS_PER_W = 4 * PR_PER_W
LW = PR_PER_W * FW


def _body(x_hbm, s_hbm, out_hbm, sbuf, xbuf, obuf):
    cid = lax.axis_index("c")
    sid = lax.axis_index("s")
    wid = sid * 2 + cid
    base_l = wid * LW
    pltpu.sync_copy(s_hbm.at[pl.ds(base_l, LW)], sbuf)
    lane4 = lax.iota(jnp.int32, 16) * 4

    def chan(c, carry):
        pltpu.sync_copy(
            x_hbm.at[pl.ds((c * H + wid * ROWS_PER_W) * W, ROWS_PER_W * W)],
            xbuf,
        )
        for i in range(LW // 16):
            s = sbuf[pl.ds(i * 16, 16)]
            row = (s >> 2) + ((i * 16) // FW) * 4
            col = (s & 3) + lane4 + ((i * 16) % FW) * 4
            obuf[pl.ds(i * 16, 16)] = plsc.load_gather(xbuf, [row * W + col])
        pltpu.sync_copy(obuf, out_hbm.at[pl.ds(c * L + base_l, LW)])
        return carry

    lax.fori_loop(0, C, chan, 0)


@jax.jit
def _run(xr, sidx):
    mesh = plsc.VectorSubcoreMesh(core_axis_name="c", subcore_axis_name="s")
    kfn = pl.kernel(
        _body,
        out_type=jax.ShapeDtypeStruct((C * L,), jnp.float32),
        mesh=mesh,
        scratch_types=[
            pltpu.VMEM((LW,), jnp.int32),
            pltpu.VMEM((ROWS_PER_W * W,), jnp.float32),
            pltpu.VMEM((LW,), jnp.float32),
        ],
        compiler_params=pltpu.CompilerParams(needs_layout_passes=False),
    )
    return kfn(xr, sidx)


def kernel(x):
    b, c, h, w = x.shape
    sidx = jax.random.randint(
        jax.random.key(42), (b, 1, 1, L), 0, 16
    ).reshape(L).astype(jnp.int32)
    xr = x.reshape(C * H * W)
    out = _run(xr, sidx)
    return out.reshape(1, C, L)


# speedup vs baseline: 2.5692x; 1.6345x over previous
import jax
import jax.numpy as jnp
from jax import lax
from jax.experimental import pallas as pl
from jax.experimental.pallas import tpu as pltpu
from jax.experimental.pallas import tpu_sc as plsc

C = 96
H = 512
W = 512
FH = 128
FW = 128
L = FH * FW
NW = 32
PR_PER_W = FH // NW
ROWS_PER_W = 4 * PR_PER_W
LW = PR_PER_W * FW
SLAB = ROWS_PER_W * W
NB = 4


def _slab_src(x_hbm, wid, c):
    return x_hbm.at[pl.ds((c * H + wid * ROWS_PER_W) * W, SLAB)]


def _body(x_hbm, s_hbm, out_hbm, sbuf, ibuf, xb0, xb1, xb2, xb3, obuf,
          load_sem):
    xbufs = (xb0, xb1, xb2, xb3)
    cid = lax.axis_index("c")
    sid = lax.axis_index("s")
    wid = sid * 2 + cid
    base_l = wid * LW

    for b in range(NB):
        pltpu.async_copy(_slab_src(x_hbm, wid, b), xbufs[b], load_sem)

    pltpu.sync_copy(s_hbm.at[pl.ds(base_l, LW)], sbuf)
    lane4 = lax.iota(jnp.int32, 16) * 4
    for i in range(LW // 16):
        s = sbuf[pl.ds(i * 16, 16)]
        row = (s >> 2) + ((i * 16) // FW) * 4
        col = (s & 3) + lane4 + ((i * 16) % FW) * 4
        ibuf[pl.ds(i * 16, 16)] = row * W + col

    def group(g, carry):
        for b in range(NB):
            c = g * NB + b
            pltpu.make_async_copy(
                _slab_src(x_hbm, wid, c), xbufs[b], load_sem
            ).wait()
            for i in range(LW // 16):
                idx = ibuf[pl.ds(i * 16, 16)]
                obuf[c, pl.ds(i * 16, 16)] = plsc.load_gather(
                    xbufs[b], [idx]
                )

            @pl.when(c + NB < C)
            def _():
                pltpu.async_copy(
                    _slab_src(x_hbm, wid, c + NB), xbufs[b], load_sem
                )

        return carry

    lax.fori_loop(0, C // NB, group, 0)
    pltpu.sync_copy(obuf, out_hbm.at[:, pl.ds(base_l, LW)])


@jax.jit
def _run(xr, sidx):
    mesh = plsc.VectorSubcoreMesh(core_axis_name="c", subcore_axis_name="s")
    kfn = pl.kernel(
        _body,
        out_type=jax.ShapeDtypeStruct((C, L), jnp.float32),
        mesh=mesh,
        scratch_types=[
            pltpu.VMEM((LW,), jnp.int32),
            pltpu.VMEM((LW,), jnp.int32),
            pltpu.VMEM((SLAB,), jnp.float32),
            pltpu.VMEM((SLAB,), jnp.float32),
            pltpu.VMEM((SLAB,), jnp.float32),
            pltpu.VMEM((SLAB,), jnp.float32),
            pltpu.VMEM((C, LW), jnp.float32),
            pltpu.SemaphoreType.DMA,
        ],
        compiler_params=pltpu.CompilerParams(needs_layout_passes=False),
    )
    return kfn(xr, sidx)


def kernel(x):
    b, c, h, w = x.shape
    sidx = jax.random.randint(
        jax.random.key(42), (b, 1, 1, L), 0, 16
    ).reshape(L).astype(jnp.int32)
    xr = x.reshape(C * H * W)
    out = _run(xr, sidx)
    return out.reshape(1, C, L)


# speedup vs baseline: 5.1517x; 2.0051x over previous
import jax
import jax.numpy as jnp
from jax import lax
from jax.experimental import pallas as pl
from jax.experimental.pallas import tpu as pltpu
from jax.experimental.pallas import tpu_sc as plsc

C = 96
H = 512
W = 512
FH = 128
FW = 128
L = FH * FW
NW = 32
PR_PER_W = FH // NW
ROWS_PER_W = 4 * PR_PER_W
LW = PR_PER_W * FW
NB = 4


def _slab_src(x_hbm, wid, c):
    return x_hbm.at[pl.ds(c * H + wid * ROWS_PER_W, ROWS_PER_W), :]


def _body(x_hbm, s_hbm, out_hbm, sbuf, rbuf, cbuf, xb0, xb1, xb2, xb3, obuf,
          load_sem):
    xbufs = (xb0, xb1, xb2, xb3)
    cid = lax.axis_index("c")
    sid = lax.axis_index("s")
    wid = sid * 2 + cid
    base_l = wid * LW

    for b in range(NB):
        pltpu.async_copy(_slab_src(x_hbm, wid, b), xbufs[b], load_sem)

    pltpu.sync_copy(s_hbm.at[pl.ds(base_l, LW)], sbuf)
    lane4 = lax.iota(jnp.int32, 16) * 4
    for i in range(LW // 16):
        s = sbuf[pl.ds(i * 16, 16)]
        rbuf[pl.ds(i * 16, 16)] = (s >> 2) + ((i * 16) // FW) * 4
        cbuf[pl.ds(i * 16, 16)] = (s & 3) + lane4 + ((i * 16) % FW) * 4

    def group(g, carry):
        for b in range(NB):
            c = g * NB + b
            pltpu.make_async_copy(
                _slab_src(x_hbm, wid, c), xbufs[b], load_sem
            ).wait()
            for i in range(LW // 16):
                row = rbuf[pl.ds(i * 16, 16)]
                col = cbuf[pl.ds(i * 16, 16)]
                obuf[c, pl.ds(i * 16, 16)] = plsc.load_gather(
                    xbufs[b], [row, col]
                )

            @pl.when(c + NB < C)
            def _():
                pltpu.async_copy(
                    _slab_src(x_hbm, wid, c + NB), xbufs[b], load_sem
                )

        return carry

    lax.fori_loop(0, C // NB, group, 0)
    pltpu.sync_copy(obuf, out_hbm.at[:, pl.ds(base_l, LW)])


@jax.jit
def _run(xr, sidx):
    mesh = plsc.VectorSubcoreMesh(core_axis_name="c", subcore_axis_name="s")
    kfn = pl.kernel(
        _body,
        out_type=jax.ShapeDtypeStruct((C, L), jnp.float32),
        mesh=mesh,
        scratch_types=[
            pltpu.VMEM((LW,), jnp.int32),
            pltpu.VMEM((LW,), jnp.int32),
            pltpu.VMEM((LW,), jnp.int32),
            pltpu.VMEM((ROWS_PER_W, W), jnp.float32),
            pltpu.VMEM((ROWS_PER_W, W), jnp.float32),
            pltpu.VMEM((ROWS_PER_W, W), jnp.float32),
            pltpu.VMEM((ROWS_PER_W, W), jnp.float32),
            pltpu.VMEM((C, LW), jnp.float32),
            pltpu.SemaphoreType.DMA,
        ],
        compiler_params=pltpu.CompilerParams(needs_layout_passes=False),
    )
    return kfn(xr, sidx)


def kernel(x):
    b, c, h, w = x.shape
    sidx = jax.random.randint(jax.random.key(42), (L,), 0, 16, jnp.int32)
    xr = x.reshape(C * H, W)
    out = _run(xr, sidx)
    return out.reshape(1, C, L)
